# baseline (speedup 1.0000x reference)
import functools
import math

import jax
import jax.numpy as jnp
from jax.experimental import pallas as pl

E = 8
TOPK = 2
D = 2048
DFF = 1408
DSH = 5632
T = 2048

FINAL_SCALE = 1.0 / math.sqrt(TOPK)


def _router_body(x_ref, rw_ref, sgw_ref, combine_ref, gate_ref):
    x = x_ref[...]
    logits = jnp.dot(x, rw_ref[...], preferred_element_type=jnp.float32)
    col = jax.lax.broadcasted_iota(jnp.int32, (T, E), 1)
    m1 = jnp.max(logits, axis=-1, keepdims=True)
    i1 = jnp.min(jnp.where(logits == m1, col, E), axis=-1, keepdims=True)
    oh1 = col == i1
    l2 = jnp.where(oh1, -jnp.inf, logits)
    m2 = jnp.max(l2, axis=-1, keepdims=True)
    i2 = jnp.min(jnp.where(l2 == m2, col, E), axis=-1, keepdims=True)
    oh2 = col == i2
    w1 = 1.0 / (1.0 + jnp.exp(m2 - m1))
    w2 = 1.0 - w1
    combine_ref[...] = jnp.where(oh1, w1, 0.0) + jnp.where(oh2, w2, 0.0)
    gate_ref[...] = jax.nn.sigmoid(
        jnp.dot(x, sgw_ref[...], preferred_element_type=jnp.float32))


def _expert_body(x_ref, gw_ref, uw_ref, dw_ref, combine_ref, out_ref, *,
                 bt):
    e = pl.program_id(0)
    f = pl.program_id(1)
    t = pl.program_id(2)
    xt = x_ref[...]
    g = jnp.dot(xt, gw_ref[0], preferred_element_type=jnp.float32)
    u = jnp.dot(xt, uw_ref[0], preferred_element_type=jnp.float32)
    h = (g * jax.nn.sigmoid(g)) * u
    part = jnp.dot(h, dw_ref[0], preferred_element_type=jnp.float32)
    c = combine_ref[pl.ds(t * bt, bt), :]
    lane = jax.lax.broadcasted_iota(jnp.int32, (bt, E), 1)
    w = jnp.sum(jnp.where(lane == e, c, 0.0), axis=1, keepdims=True)
    part = part * w

    first = jnp.logical_and(e == 0, f == 0)

    @pl.when(first)
    def _init():
        out_ref[pl.ds(t * bt, bt), :] = part

    @pl.when(jnp.logical_not(first))
    def _acc():
        out_ref[pl.ds(t * bt, bt), :] += part


def _shared_body(x_ref, gw_ref, uw_ref, dw_ref, eacc_ref, gate_ref, out_ref,
                 *, bt, ns):
    s = pl.program_id(0)
    t = pl.program_id(1)
    rows = pl.ds(t * bt, bt)
    xt = x_ref[...]
    g = jnp.dot(xt, gw_ref[...], preferred_element_type=jnp.float32)
    u = jnp.dot(xt, uw_ref[...], preferred_element_type=jnp.float32)
    h = (g * jax.nn.sigmoid(g)) * u
    part = jnp.dot(h, dw_ref[...], preferred_element_type=jnp.float32)

    @pl.when(s == 0)
    def _init():
        out_ref[rows, :] = part

    @pl.when(s != 0)
    def _acc():
        out_ref[rows, :] += part

    @pl.when(s == ns - 1)
    def _finalize():
        gate = gate_ref[...]
        out_ref[rows, :] = (
            eacc_ref[...] + gate * out_ref[rows, :]) * FINAL_SCALE


def kernel(hidden_states, expert_gate_w, expert_up_w, expert_down_w,
           shared_gate_w, shared_up_w, shared_down_w, router_w,
           shared_expert_gate_w):
    x = hidden_states.reshape(T, D)

    combine, gate = pl.pallas_call(
        _router_body,
        out_shape=(
            jax.ShapeDtypeStruct((T, E), jnp.float32),
            jax.ShapeDtypeStruct((T, 1), jnp.float32),
        ),
    )(x, router_w, shared_expert_gate_w)

    BT = 256
    BF = 128
    NF = DFF // BF
    eacc = pl.pallas_call(
        functools.partial(_expert_body, bt=BT),
        grid=(E, NF, T // BT),
        in_specs=[
            pl.BlockSpec((BT, D), lambda e, f, t: (t, 0)),
            pl.BlockSpec((1, D, BF), lambda e, f, t: (e, 0, f)),
            pl.BlockSpec((1, D, BF), lambda e, f, t: (e, 0, f)),
            pl.BlockSpec((1, BF, D), lambda e, f, t: (e, f, 0)),
            pl.BlockSpec((T, E), lambda e, f, t: (0, 0)),
        ],
        out_specs=pl.BlockSpec((T, D), lambda e, f, t: (0, 0)),
        out_shape=jax.ShapeDtypeStruct((T, D), jnp.float32),
    )(x, expert_gate_w, expert_up_w, expert_down_w, combine)

    BS = 512
    NS = DSH // BS
    out = pl.pallas_call(
        functools.partial(_shared_body, bt=BT, ns=NS),
        grid=(NS, T // BT),
        in_specs=[
            pl.BlockSpec((BT, D), lambda s, t: (t, 0)),
            pl.BlockSpec((D, BS), lambda s, t: (0, s)),
            pl.BlockSpec((D, BS), lambda s, t: (0, s)),
            pl.BlockSpec((BS, D), lambda s, t: (s, 0)),
            pl.BlockSpec((BT, D), lambda s, t: (t, 0)),
            pl.BlockSpec((BT, 1), lambda s, t: (t, 0)),
        ],
        out_specs=pl.BlockSpec((T, D), lambda s, t: (0, 0)),
        out_shape=jax.ShapeDtypeStruct((T, D), jnp.float32),
    )(x, shared_gate_w, shared_up_w, shared_down_w, eacc, gate)

    return out
